# baseline (speedup 1.0000x reference)
import jax
import jax.numpy as jnp
from jax.experimental import pallas as pl
from jax.experimental.pallas import tpu as pltpu


def _body(x_hbm, w_hbm, b_hbm, o_ref, xbuf, wbuf, bbuf, sx, sw, sb):
    cx = pltpu.make_async_copy(x_hbm, xbuf, sx)
    cx.start()
    cw = pltpu.make_async_copy(w_hbm, wbuf, sw)
    cw.start()
    cb = pltpu.make_async_copy(b_hbm, bbuf, sb)
    cb.start()
    cw.wait()
    cb.wait()
    w64 = wbuf[...]
    r = jax.lax.broadcasted_iota(jnp.int32, (128, 64), 0)
    c = jax.lax.broadcasted_iota(jnp.int32, (128, 64), 1)
    a_even = jnp.where(r == 2 * c, 1.0, 0.0).astype(jnp.float32)
    a_odd = jnp.where(r == 2 * c + 1, 1.0, 0.0).astype(jnp.float32)
    dn = (((1,), (0,)), ((), ()))
    w = (jax.lax.dot_general(a_even, w64[:, :64], dn,
                             preferred_element_type=jnp.float32)
         + jax.lax.dot_general(a_odd, w64[:, 64:], dn,
                               preferred_element_type=jnp.float32))
    cx.wait()
    y = jax.lax.dot_general(jnp.maximum(xbuf[...], 0.0), w, dn,
                            preferred_element_type=jnp.float32)
    o_ref[...] = y + bbuf[...]


def kernel(x_subject, x_region, edge_index_sr, edge_index_rr, edge_attr_sr,
           edge_attr_rr, sage_Wl0, sage_bl0, sage_Wr0, gcn_W0, gcn_b0,
           sage_Wl1, sage_bl1, sage_Wr1, gcn_W1, gcn_b1, lin_W, lin_b):
    m, d = x_subject.shape
    out_dim = lin_W.shape[1]
    return pl.pallas_call(
        _body,
        in_specs=[
            pl.BlockSpec(memory_space=pltpu.MemorySpace.HBM),
            pl.BlockSpec(memory_space=pltpu.MemorySpace.HBM),
            pl.BlockSpec(memory_space=pltpu.MemorySpace.HBM),
        ],
        out_specs=pl.BlockSpec(memory_space=pltpu.MemorySpace.VMEM),
        out_shape=jax.ShapeDtypeStruct((m, out_dim), jnp.float32),
        scratch_shapes=[
            pltpu.VMEM((m, d), jnp.float32),
            pltpu.VMEM((64, 128), jnp.float32),
            pltpu.VMEM((1, 64), jnp.float32),
            pltpu.SemaphoreType.DMA,
            pltpu.SemaphoreType.DMA,
            pltpu.SemaphoreType.DMA,
        ],
    )(x_subject, lin_W.reshape(out_dim, d), lin_b.reshape(1, 64))


# speedup vs baseline: 1.0254x; 1.0254x over previous
import jax
import jax.numpy as jnp
from jax.experimental import pallas as pl


def _relu_matmul_bias_kernel(x_ref, w_ref, b_ref, o_ref):
    x = jnp.maximum(x_ref[...], 0.0)
    acc = jax.lax.dot_general(
        x, w_ref[...], (((1,), (0,)), ((), ())),
        preferred_element_type=jnp.float32,
    )
    o_ref[...] = acc + b_ref[...]


def kernel(x_subject, x_region, edge_index_sr, edge_index_rr, edge_attr_sr,
           edge_attr_rr, sage_Wl0, sage_bl0, sage_Wr0, gcn_W0, gcn_b0,
           sage_Wl1, sage_bl1, sage_Wr1, gcn_W1, gcn_b1, lin_W, lin_b):
    m, d = x_subject.shape
    out_dim = lin_W.shape[1]
    return pl.pallas_call(
        _relu_matmul_bias_kernel,
        out_shape=jax.ShapeDtypeStruct((m, out_dim), jnp.float32),
    )(x_subject, lin_W, lin_b.reshape(1, out_dim))
